# baseline (speedup 1.0000x reference)
import functools

import jax
import jax.numpy as jnp
from jax import lax
from jax.experimental import pallas as pl
from jax.experimental.pallas import tpu as pltpu
from jax.experimental.pallas import tpu_sc as plsc

_NUM_MOLECULES = 16384
_NUM_ATOMS = 200

_NC = 2
_NS = 16
_NW = _NC * _NS
_LANES = 16
_MOLS_PER_W = _NUM_MOLECULES // _NW
_TILE_MOLS = 128
_N_TILES = _MOLS_PER_W // _TILE_MOLS
_VECS = _TILE_MOLS // _LANES
_QUAD = 4
_NSPEC = 8


def _shift_kernel(spec_hbm, en_hbm, tab_hbm, out_hbm,
                  spec_v, en_v, out_v, tab_v, tab4_v,
                  lsem0, lsem1, lsem2, lsem3):
    lsems = (lsem0, lsem1, lsem2, lsem3)
    wid = lax.axis_index("s") * _NC + lax.axis_index("c")
    mol0 = wid * _MOLS_PER_W

    def panel_slice(t):
        return pl.ds(mol0 + t * _TILE_MOLS, _TILE_MOLS)

    def start_load(t):
        return pltpu.async_copy(spec_hbm.at[:, panel_slice(t)],
                                spec_v.at[t], lsems[t])

    loads = [start_load(t) for t in range(_N_TILES)]

    pltpu.sync_copy(tab_hbm, tab_v.at[pl.ds(0, _NSPEC)])
    pltpu.sync_copy(en_hbm.at[pl.ds(mol0, _MOLS_PER_W)], en_v)

    lane = lax.iota(jnp.int32, _LANES)
    gd = plsc.load_gather(tab_v, [lane & 7])
    chalf = lane >> 3

    def tab_body(k, _):
        a = lax.shift_right_logical(k, 5)
        b = lax.shift_right_logical(k, 2) & 7
        c0 = (k & 3) * 2
        ga = plsc.load_gather(tab_v, [jnp.broadcast_to(a, (_LANES,))])
        gb = plsc.load_gather(tab_v, [jnp.broadcast_to(b, (_LANES,))])
        gc = plsc.load_gather(tab_v, [jnp.broadcast_to(c0, (_LANES,)) + chalf])
        tab4_v[pl.ds(k * _LANES, _LANES)] = (ga + gb) + (gc + gd)
        return 0

    lax.fori_loop(0, _NSPEC ** _QUAD // _LANES, tab_body, 0)

    def compute(t):
        def quad_body(q, accs):
            a = q * _QUAD
            new = []
            for j in range(_VECS):
                sl = pl.ds(j * _LANES, _LANES)
                s0 = spec_v[t, a, sl]
                s1 = spec_v[t, a + 1, sl]
                s2 = spec_v[t, a + 2, sl]
                s3 = spec_v[t, a + 3, sl]
                idx = ((lax.shift_left(s0, 9) | lax.shift_left(s1, 6))
                       | (lax.shift_left(s2, 3) | s3))
                new.append(accs[j] + plsc.load_gather(tab4_v, [idx]))
            return tuple(new)

        accs = lax.fori_loop(
            0, _NUM_ATOMS // _QUAD, quad_body,
            tuple(jnp.zeros((_LANES,), jnp.float32) for _ in range(_VECS)))
        for j in range(_VECS):
            sl = pl.ds(t * _TILE_MOLS + j * _LANES, _LANES)
            out_v[sl] = accs[j] + en_v[sl]

    for t in range(_N_TILES):
        loads[t].wait()
        compute(t)

    pltpu.sync_copy(out_v, out_hbm.at[pl.ds(mol0, _MOLS_PER_W)])


@jax.jit
def _shifted(spec_t, energies, self_energies):
    mesh = plsc.VectorSubcoreMesh(core_axis_name="c", subcore_axis_name="s")
    call = functools.partial(
        pl.kernel,
        out_type=jax.ShapeDtypeStruct((_NUM_MOLECULES,), jnp.float32),
        mesh=mesh,
        scratch_types=[
            pltpu.VMEM((_N_TILES, _NUM_ATOMS, _TILE_MOLS), jnp.int32),
            pltpu.VMEM((_MOLS_PER_W,), jnp.float32),
            pltpu.VMEM((_MOLS_PER_W,), jnp.float32),
            pltpu.VMEM((_LANES,), jnp.float32),
            pltpu.VMEM((_NSPEC ** _QUAD,), jnp.float32),
        ] + [pltpu.SemaphoreType.DMA] * 4,
        compiler_params=pltpu.CompilerParams(needs_layout_passes=False,
                                             use_tc_tiling_on_sc=True),
    )(_shift_kernel)
    return call(spec_t, energies, self_energies)


def _copy_body(src_ref, dst_ref):
    dst_ref[...] = src_ref[...]


@jax.jit
def _passthrough(spec_t):
    grid = _NUM_ATOMS // 8
    return pl.pallas_call(
        _copy_body,
        out_shape=jax.ShapeDtypeStruct((_NUM_ATOMS, _NUM_MOLECULES),
                                       jnp.int32),
        grid=(grid,),
        in_specs=[pl.BlockSpec((8, _NUM_MOLECULES), lambda i: (i, 0))],
        out_specs=pl.BlockSpec((8, _NUM_MOLECULES), lambda i: (i, 0)),
    )(spec_t)


def kernel(species, energies, self_energies):
    spec_t = species.T
    shifted = _shifted(spec_t, energies, self_energies)
    spec_out = _passthrough(spec_t)
    return spec_out.T, shifted


# speedup vs baseline: 1.0975x; 1.0975x over previous
import functools

import jax
import jax.numpy as jnp
from jax import lax
from jax.experimental import pallas as pl
from jax.experimental.pallas import tpu as pltpu
from jax.experimental.pallas import tpu_sc as plsc

_NUM_MOLECULES = 16384
_NUM_ATOMS = 200

_NC = 2
_NS = 16
_NW = _NC * _NS
_LANES = 16
_MOLS_PER_W = _NUM_MOLECULES // _NW
_TILE_MOLS = 128
_N_TILES = _MOLS_PER_W // _TILE_MOLS
_VECS = _TILE_MOLS // _LANES
_QUAD = 4
_NSPEC = 8


def _shift_kernel(spec_hbm, en_hbm, tab_hbm, out_hbm, spec_out_hbm,
                  spec_v, en_v, out_v, tab_v, tab4_v,
                  lsem0, lsem1, lsem2, lsem3, wsem0, wsem1, wsem2, wsem3):
    lsems = (lsem0, lsem1, lsem2, lsem3)
    wsems = (wsem0, wsem1, wsem2, wsem3)
    wid = lax.axis_index("s") * _NC + lax.axis_index("c")
    mol0 = wid * _MOLS_PER_W

    def panel_slice(t):
        return pl.ds(mol0 + t * _TILE_MOLS, _TILE_MOLS)

    def start_load(t):
        return pltpu.async_copy(spec_hbm.at[:, panel_slice(t)],
                                spec_v.at[t], lsems[t])

    def start_write(t):
        return pltpu.async_copy(spec_v.at[t],
                                spec_out_hbm.at[:, panel_slice(t)], wsems[t])

    loads = [start_load(t) for t in range(_N_TILES)]

    pltpu.sync_copy(tab_hbm, tab_v.at[pl.ds(0, _NSPEC)])
    pltpu.sync_copy(en_hbm.at[pl.ds(mol0, _MOLS_PER_W)], en_v)

    lane = lax.iota(jnp.int32, _LANES)
    gd = plsc.load_gather(tab_v, [lane & 7])
    chalf = lane >> 3

    def tab_body(k, _):
        a = lax.shift_right_logical(k, 5)
        b = lax.shift_right_logical(k, 2) & 7
        c0 = (k & 3) * 2
        ga = plsc.load_gather(tab_v, [jnp.broadcast_to(a, (_LANES,))])
        gb = plsc.load_gather(tab_v, [jnp.broadcast_to(b, (_LANES,))])
        gc = plsc.load_gather(tab_v, [jnp.broadcast_to(c0, (_LANES,)) + chalf])
        tab4_v[pl.ds(k * _LANES, _LANES)] = (ga + gb) + (gc + gd)
        return 0

    lax.fori_loop(0, _NSPEC ** _QUAD // _LANES, tab_body, 0)

    def compute(t):
        def quad_body(q, accs):
            a = q * _QUAD
            new = []
            for j in range(_VECS):
                sl = pl.ds(j * _LANES, _LANES)
                s0 = spec_v[t, a, sl]
                s1 = spec_v[t, a + 1, sl]
                s2 = spec_v[t, a + 2, sl]
                s3 = spec_v[t, a + 3, sl]
                idx = ((lax.shift_left(s0, 9) | lax.shift_left(s1, 6))
                       | (lax.shift_left(s2, 3) | s3))
                new.append(accs[j] + plsc.load_gather(tab4_v, [idx]))
            return tuple(new)

        accs = lax.fori_loop(
            0, _NUM_ATOMS // _QUAD, quad_body,
            tuple(jnp.zeros((_LANES,), jnp.float32) for _ in range(_VECS)))
        for j in range(_VECS):
            sl = pl.ds(t * _TILE_MOLS + j * _LANES, _LANES)
            out_v[sl] = accs[j] + en_v[sl]

    writes = []
    for t in range(_N_TILES):
        loads[t].wait()
        writes.append(start_write(t))
        compute(t)
    for w in writes:
        w.wait()

    pltpu.sync_copy(out_v, out_hbm.at[pl.ds(mol0, _MOLS_PER_W)])


@jax.jit
def _shifted(spec_t, energies, self_energies):
    mesh = plsc.VectorSubcoreMesh(core_axis_name="c", subcore_axis_name="s")
    call = functools.partial(
        pl.kernel,
        out_type=[
            jax.ShapeDtypeStruct((_NUM_MOLECULES,), jnp.float32),
            jax.ShapeDtypeStruct((_NUM_ATOMS, _NUM_MOLECULES), jnp.int32),
        ],
        mesh=mesh,
        scratch_types=[
            pltpu.VMEM((_N_TILES, _NUM_ATOMS, _TILE_MOLS), jnp.int32),
            pltpu.VMEM((_MOLS_PER_W,), jnp.float32),
            pltpu.VMEM((_MOLS_PER_W,), jnp.float32),
            pltpu.VMEM((_LANES,), jnp.float32),
            pltpu.VMEM((_NSPEC ** _QUAD,), jnp.float32),
        ] + [pltpu.SemaphoreType.DMA] * 8,
        compiler_params=pltpu.CompilerParams(needs_layout_passes=False,
                                             use_tc_tiling_on_sc=True),
    )(_shift_kernel)
    return call(spec_t, energies, self_energies)


def kernel(species, energies, self_energies):
    shifted, spec_out = _shifted(species.T, energies, self_energies)
    return spec_out.T, shifted
